# baseline (speedup 1.0000x reference)
import functools

import jax
import jax.numpy as jnp
from jax import lax
from jax.experimental import pallas as pl
from jax.experimental.pallas import tpu as pltpu
from jax.experimental.pallas import tpu_sc as plsc

_SPACING = 0.001
_SHIFT = _SPACING / 2.0

_N_IN = 4096
_C = 128
_N_NEW = 4096

_NG = 4
_UNROLL = 32


def _argmin_body(qx_ref, qy_ref, c_ref, idx_ref):
    shape = (8, 128)
    qx = [qx_ref[g] for g in range(_NG)]
    qy = [qy_ref[g] for g in range(_NG)]
    inf = jnp.full(shape, jnp.inf, jnp.float32)
    zero = jnp.zeros(shape, jnp.int32)
    ns = 4
    state = tuple([inf] * _NG for _ in range(ns)) + tuple(
        [zero] * _NG for _ in range(ns)
    )

    def body(k, st):
        best = [list(b) for b in st[:ns]]
        bidx = [list(b) for b in st[ns:]]
        j0 = k * _UNROLL
        for u in range(_UNROLL):
            s = u % ns
            j = j0 + u
            cx = jnp.full(shape, c_ref[0, j])
            cy = jnp.full(shape, c_ref[1, j])
            for g in range(_NG):
                dx = qx[g] - cx
                dy = qy[g] - cy
                d2 = dx * dx + dy * dy
                lt = d2 < best[s][g]
                best[s][g] = jnp.where(lt, d2, best[s][g])
                bidx[s][g] = jnp.where(lt, j, bidx[s][g])
        return tuple(best) + tuple(bidx)

    st = lax.fori_loop(0, _N_IN // _UNROLL, body, state)
    best, bidx = st[:ns], st[ns:]
    for g in range(_NG):
        bv, bi = best[0][g], bidx[0][g]
        for s in range(1, ns):
            v, i = best[s][g], bidx[s][g]
            take = (v < bv) | ((v == bv) & (i < bi))
            bv = jnp.where(take, v, bv)
            bi = jnp.where(take, i, bi)
        idx_ref[g] = bi


def _nn_argmin(qx, qy, coords_t, interpret=False):
    return pl.pallas_call(
        _argmin_body,
        in_specs=[
            pl.BlockSpec(memory_space=pltpu.VMEM),
            pl.BlockSpec(memory_space=pltpu.VMEM),
            pl.BlockSpec(memory_space=pltpu.SMEM),
        ],
        out_specs=pl.BlockSpec(memory_space=pltpu.VMEM),
        out_shape=jax.ShapeDtypeStruct((_NG, 8, 128), jnp.int32),
        interpret=interpret,
    )(qx, qy, coords_t).reshape(_N_NEW)


_NC, _NS = 2, 16
_NW = _NC * _NS
_R_PER_W = _C // _NW
_L = 16


def _gather_body(values_hbm, idx_hbm, out_hbm, idx_v, rows_v, new_v, sem, lsem):
    wid = lax.axis_index("s") * _NC + lax.axis_index("c")
    row0 = wid * _R_PER_W
    copies = [pltpu.make_async_copy(idx_hbm, idx_v, sem)]
    copies += [
        pltpu.make_async_copy(
            values_hbm.at[row0 + r], rows_v.at[pl.ds(r * _N_IN, _N_IN)], sem
        )
        for r in range(_R_PER_W)
    ]
    for cp in copies:
        cp.start()
    for cp in copies:
        cp.wait()

    left = [
        pltpu.make_async_copy(
            rows_v.at[pl.ds(r * _N_IN, _N_IN)],
            out_hbm.at[row0 + r, pl.ds(0, _N_IN)],
            lsem,
        )
        for r in range(_R_PER_W)
    ]
    for cp in left:
        cp.start()

    gunroll = 4

    def body(k, carry):
        for u in range(gunroll):
            off = (k * gunroll + u) * _L
            ich = idx_v[pl.ds(off, _L)]
            for r in range(_R_PER_W):
                g = plsc.load_gather(rows_v, [ich + (r * _N_IN)])
                new_v[pl.ds(r * _N_IN + off, _L)] = g
        return carry

    lax.fori_loop(0, _N_IN // (_L * gunroll), body, 0)

    outs = [
        pltpu.make_async_copy(
            new_v.at[pl.ds(r * _N_IN, _N_IN)],
            out_hbm.at[row0 + r, pl.ds(_N_IN, _N_IN)],
            sem,
        )
        for r in range(_R_PER_W)
    ]
    for cp in outs:
        cp.start()
    for cp in outs:
        cp.wait()
    for cp in left:
        cp.wait()


@functools.cache
def _make_gather():
    return pl.kernel(
        _gather_body,
        out_type=jax.ShapeDtypeStruct((_C, 2 * _N_IN), jnp.float32),
        mesh=plsc.VectorSubcoreMesh(
            core_axis_name="c", subcore_axis_name="s", num_cores=_NC
        ),
        scratch_types=[
            pltpu.VMEM((_N_NEW,), jnp.int32),
            pltpu.VMEM((_R_PER_W * _N_IN,), jnp.float32),
            pltpu.VMEM((_R_PER_W * _N_IN,), jnp.float32),
            pltpu.SemaphoreType.DMA,
            pltpu.SemaphoreType.DMA,
        ],
        compiler_params=pltpu.CompilerParams(needs_layout_passes=False),
    )


@jax.jit
def kernel(coords, values, dropped_coords):
    q = dropped_coords - _SHIFT
    qx = q[:, 0].reshape(_NG, 8, 128)
    qy = q[:, 1].reshape(_NG, 8, 128)
    nn_idx = _nn_argmin(qx, qy, coords.T)
    return _make_gather()(values, nn_idx)


# speedup vs baseline: 1.0084x; 1.0084x over previous
import functools

import jax
import jax.numpy as jnp
from jax import lax
from jax.experimental import pallas as pl
from jax.experimental.pallas import tpu as pltpu
from jax.experimental.pallas import tpu_sc as plsc

_SPACING = 0.001
_SHIFT = _SPACING / 2.0

_N_IN = 4096
_C = 128
_N_NEW = 4096

_NG = 4
_UNROLL = 32


def _argmin_body(qx_ref, qy_ref, c_ref, idx_ref):
    shape = (8, 128)
    qx = [qx_ref[g] for g in range(_NG)]
    qy = [qy_ref[g] for g in range(_NG)]
    inf = jnp.full(shape, jnp.inf, jnp.float32)
    zero = jnp.zeros(shape, jnp.int32)
    ns = 2
    state = tuple([inf] * _NG for _ in range(ns)) + tuple(
        [zero] * _NG for _ in range(ns)
    )

    def body(k, st):
        best = [list(b) for b in st[:ns]]
        bidx = [list(b) for b in st[ns:]]
        j0 = k * _UNROLL
        for u in range(_UNROLL):
            s = u % ns
            j = j0 + u
            cx = jnp.full(shape, c_ref[0, j])
            cy = jnp.full(shape, c_ref[1, j])
            for g in range(_NG):
                dx = qx[g] - cx
                dy = qy[g] - cy
                d2 = dx * dx + dy * dy
                lt = d2 < best[s][g]
                best[s][g] = jnp.where(lt, d2, best[s][g])
                bidx[s][g] = jnp.where(lt, j, bidx[s][g])
        return tuple(best) + tuple(bidx)

    st = lax.fori_loop(0, _N_IN // _UNROLL, body, state)
    best, bidx = st[:ns], st[ns:]
    for g in range(_NG):
        bv, bi = best[0][g], bidx[0][g]
        for s in range(1, ns):
            v, i = best[s][g], bidx[s][g]
            take = (v < bv) | ((v == bv) & (i < bi))
            bv = jnp.where(take, v, bv)
            bi = jnp.where(take, i, bi)
        idx_ref[g] = bi


def _nn_argmin(qx, qy, coords_t, interpret=False):
    return pl.pallas_call(
        _argmin_body,
        in_specs=[
            pl.BlockSpec(memory_space=pltpu.VMEM),
            pl.BlockSpec(memory_space=pltpu.VMEM),
            pl.BlockSpec(memory_space=pltpu.SMEM),
        ],
        out_specs=pl.BlockSpec(memory_space=pltpu.VMEM),
        out_shape=jax.ShapeDtypeStruct((_NG, 8, 128), jnp.int32),
        interpret=interpret,
    )(qx, qy, coords_t).reshape(_N_NEW)


_NC, _NS = 2, 16
_NW = _NC * _NS
_R_PER_W = _C // _NW
_L = 16


def _gather_body(values_hbm, idx_hbm, out_hbm, idx_v, rows_v, new_v, sem, lsem):
    wid = lax.axis_index("s") * _NC + lax.axis_index("c")
    row0 = wid * _R_PER_W
    copies = [pltpu.make_async_copy(idx_hbm, idx_v, sem)]
    copies += [
        pltpu.make_async_copy(
            values_hbm.at[row0 + r], rows_v.at[pl.ds(r * _N_IN, _N_IN)], sem
        )
        for r in range(_R_PER_W)
    ]
    for cp in copies:
        cp.start()
    for cp in copies:
        cp.wait()

    left = [
        pltpu.make_async_copy(
            rows_v.at[pl.ds(r * _N_IN, _N_IN)],
            out_hbm.at[row0 + r, pl.ds(0, _N_IN)],
            lsem,
        )
        for r in range(_R_PER_W)
    ]
    for cp in left:
        cp.start()

    gunroll = 4

    def body(k, carry):
        for u in range(gunroll):
            off = (k * gunroll + u) * _L
            ich = idx_v[pl.ds(off, _L)]
            for r in range(_R_PER_W):
                g = plsc.load_gather(rows_v, [ich + (r * _N_IN)])
                new_v[pl.ds(r * _N_IN + off, _L)] = g
        return carry

    lax.fori_loop(0, _N_IN // (_L * gunroll), body, 0)

    outs = [
        pltpu.make_async_copy(
            new_v.at[pl.ds(r * _N_IN, _N_IN)],
            out_hbm.at[row0 + r, pl.ds(_N_IN, _N_IN)],
            sem,
        )
        for r in range(_R_PER_W)
    ]
    for cp in outs:
        cp.start()
    for cp in outs:
        cp.wait()
    for cp in left:
        cp.wait()


@functools.cache
def _make_gather():
    return pl.kernel(
        _gather_body,
        out_type=jax.ShapeDtypeStruct((_C, 2 * _N_IN), jnp.float32),
        mesh=plsc.VectorSubcoreMesh(
            core_axis_name="c", subcore_axis_name="s", num_cores=_NC
        ),
        scratch_types=[
            pltpu.VMEM((_N_NEW,), jnp.int32),
            pltpu.VMEM((_R_PER_W * _N_IN,), jnp.float32),
            pltpu.VMEM((_R_PER_W * _N_IN,), jnp.float32),
            pltpu.SemaphoreType.DMA,
            pltpu.SemaphoreType.DMA,
        ],
        compiler_params=pltpu.CompilerParams(needs_layout_passes=False),
    )


@jax.jit
def kernel(coords, values, dropped_coords):
    q = dropped_coords - _SHIFT
    qx = q[:, 0].reshape(_NG, 8, 128)
    qy = q[:, 1].reshape(_NG, 8, 128)
    nn_idx = _nn_argmin(qx, qy, coords.T)
    return _make_gather()(values, nn_idx)
